# baseline (speedup 1.0000x reference)
import dataclasses
import functools

import jax
import jax.numpy as jnp
from jax import lax
from jax.experimental import pallas as pl
from jax.experimental.pallas import tpu as pltpu
from jax.experimental.pallas import tpu_sc as plsc

EPS = 1e-8
NUM_CORES = 2
NUM_SUBCORES = 16
NW = NUM_CORES * NUM_SUBCORES
NPAD = 10240
GC = 80


def _prep_body(n_rows, p_ref, c_ref, t_ref, src_ref, dst_ref, s_ref, b_ref,
               tc_ref, pn_ref, cn_ref, ex_ref, fx_ref):
    scale = s_ref[0, 0]
    p = p_ref[...]
    pn = jnp.maximum(jnp.sqrt(jnp.sum(p * p, axis=1, keepdims=True)), EPS)
    pn_ref[...] = p * (scale / pn)
    c = c_ref[...]
    cn = jnp.maximum(jnp.sqrt(jnp.sum(c * c, axis=1, keepdims=True)), EPS)
    n = c.shape[0]
    cn_ref[0:n, :] = c / cn
    cn_ref[n:NPAD, :] = jnp.zeros((NPAD - n, c.shape[1]), jnp.float32)
    t = t_ref[...]
    boost = jnp.where(t > 0, 1.0 / (t + 1.0), jnp.zeros_like(t))
    ex_ref[...] = tc_ref[0, 0] * boost + b_ref[0, 0]
    src = src_ref[...]
    dst = dst_ref[...]
    fx_ref[...] = ((dst >> 7) * n_rows + src) * 128 + (dst & 127)


def _mm_body(pn_ref, cn_ref, o_ref):
    res = lax.dot_general(
        pn_ref[...], cn_ref[...], (((1,), (1,)), ((), ())),
        preferred_element_type=jnp.float32)
    o_ref[...] = res[None, :, :]


def _sc_body(epw, ng, s_hbm, fx_hbm, ex_hbm, out_hbm, fv, ev, vals, sem):
    wid = lax.axis_index("c") * NUM_SUBCORES + lax.axis_index("s")
    base = wid * epw

    pltpu.sync_copy(fx_hbm.at[pl.ds(base, epw)], fv)
    pltpu.sync_copy(ex_hbm.at[pl.ds(base, epw)], ev)

    @pl.loop(0, ng)
    def _issue(k):
        sl = pl.ds(k * GC, GC)
        pltpu.async_copy(s_hbm.at[fv.at[sl]], vals.at[sl], sem)

    @pl.loop(0, ng)
    def _drain(k):
        pltpu.make_async_copy(
            s_hbm.at[pl.ds(0, GC)], vals.at[pl.ds(0, GC)], sem).wait()

    @pl.loop(0, epw // 16)
    def _add(g):
        sl = pl.ds(g * 16, 16)
        vals[sl] = vals[sl] + ev[sl]

    pltpu.sync_copy(vals, out_hbm.at[pl.ds(base, epw)])


def kernel(patient_embeds, condition_embeds, edge_index, tte, scale, bias,
           time_coeff):
    n, d = patient_embeds.shape
    e = edge_index.shape[1]
    assert d == 128 and n == 10000 and e % (NW * GC) == 0

    pn, cnp, ex, fx = pl.pallas_call(
        functools.partial(_prep_body, n),
        out_shape=(
            jax.ShapeDtypeStruct((n, d), jnp.float32),
            jax.ShapeDtypeStruct((NPAD, d), jnp.float32),
            jax.ShapeDtypeStruct((e // 128, 128), jnp.float32),
            jax.ShapeDtypeStruct((e // 128, 128), jnp.int32),
        ),
        in_specs=[
            pl.BlockSpec(memory_space=pltpu.VMEM),
            pl.BlockSpec(memory_space=pltpu.VMEM),
            pl.BlockSpec(memory_space=pltpu.VMEM),
            pl.BlockSpec(memory_space=pltpu.VMEM),
            pl.BlockSpec(memory_space=pltpu.VMEM),
            pl.BlockSpec(memory_space=pltpu.SMEM),
            pl.BlockSpec(memory_space=pltpu.SMEM),
            pl.BlockSpec(memory_space=pltpu.SMEM),
        ],
    )(patient_embeds, condition_embeds,
      tte.reshape(e // 128, 128),
      edge_index[0].reshape(e // 128, 128),
      edge_index[1].reshape(e // 128, 128),
      scale.reshape(1, 1).astype(jnp.float32),
      bias.reshape(1, 1).astype(jnp.float32),
      time_coeff.reshape(1, 1).astype(jnp.float32))
    ex = ex.reshape(e)
    fx = fx.reshape(e)

    bi = 1000
    nkc = NPAD // d
    s = pl.pallas_call(
        _mm_body,
        grid=(n // bi, nkc),
        out_shape=jax.ShapeDtypeStruct((nkc, n, d), jnp.float32),
        in_specs=[
            pl.BlockSpec((bi, d), lambda i, k: (i, 0)),
            pl.BlockSpec((d, d), lambda i, k: (k, 0)),
        ],
        out_specs=pl.BlockSpec((1, bi, d), lambda i, k: (k, i, 0)),
    )(pn, cnp)
    s_flat = s.reshape(nkc * n * d)

    epw = e // NW
    ng = epw // GC
    mesh = plsc.VectorSubcoreMesh(core_axis_name="c", subcore_axis_name="s")
    cp = pltpu.CompilerParams()
    if "needs_layout_passes" in pltpu.CompilerParams.__dataclass_fields__:
        cp = dataclasses.replace(cp, needs_layout_passes=False)
    sc = pl.kernel(
        functools.partial(_sc_body, epw, ng),
        out_type=jax.ShapeDtypeStruct((e,), jnp.float32),
        mesh=mesh,
        scratch_types=[
            pltpu.VMEM((epw,), jnp.int32),
            pltpu.VMEM((epw,), jnp.float32),
            pltpu.VMEM((epw,), jnp.float32),
            pltpu.SemaphoreType.DMA,
        ],
        compiler_params=cp,
    )
    return sc(s_flat, fx, ex)


# speedup vs baseline: 2.7108x; 2.7108x over previous
import dataclasses
import functools

import jax
import jax.numpy as jnp
from jax import lax
from jax.experimental import pallas as pl
from jax.experimental.pallas import tpu as pltpu
from jax.experimental.pallas import tpu_sc as plsc

EPS = 1e-8
NUM_CORES = 2
NUM_SUBCORES = 16
NW = NUM_CORES * NUM_SUBCORES
NPAD = 10240
GC = 80


def _prep_body(n_rows, p_ref, c_ref, t_ref, src_ref, dst_ref, s_ref, b_ref,
               tc_ref, pn_ref, cn_ref, ex_ref, fx_ref):
    scale = s_ref[0, 0]
    p = p_ref[...]
    pn = jnp.maximum(jnp.sqrt(jnp.sum(p * p, axis=1, keepdims=True)), EPS)
    pn_ref[...] = p * (scale / pn)
    c = c_ref[...]
    cn = jnp.maximum(jnp.sqrt(jnp.sum(c * c, axis=1, keepdims=True)), EPS)
    n = c.shape[0]
    cn_ref[0:n, :] = c / cn
    cn_ref[n:NPAD, :] = jnp.zeros((NPAD - n, c.shape[1]), jnp.float32)
    t = t_ref[...]
    boost = jnp.where(t > 0, 1.0 / (t + 1.0), jnp.zeros_like(t))
    ex_ref[...] = tc_ref[0, 0] * boost + b_ref[0, 0]
    src = src_ref[...]
    dst = dst_ref[...]
    fx_ref[...] = ((dst >> 7) * n_rows + src) * 128 + (dst & 127)


def _mm_body(pn_ref, cn_ref, o_ref):
    res = lax.dot_general(
        pn_ref[...], cn_ref[...], (((1,), (1,)), ((), ())),
        preferred_element_type=jnp.float32)
    for kk in range(o_ref.shape[0]):
        o_ref[kk] = res[:, kk * 128:(kk + 1) * 128]


def _sc_body(epw, ng, s_hbm, fx_hbm, ex_hbm, out_hbm, fv, ev, vals, sem):
    wid = lax.axis_index("c") * NUM_SUBCORES + lax.axis_index("s")
    base = wid * epw

    pltpu.sync_copy(fx_hbm.at[pl.ds(base, epw)], fv)
    pltpu.sync_copy(ex_hbm.at[pl.ds(base, epw)], ev)

    @pl.loop(0, ng)
    def _issue(k):
        sl = pl.ds(k * GC, GC)
        pltpu.async_copy(s_hbm.at[fv.at[sl]], vals.at[sl], sem)

    @pl.loop(0, ng)
    def _drain(k):
        pltpu.make_async_copy(
            s_hbm.at[pl.ds(0, GC)], vals.at[pl.ds(0, GC)], sem).wait()

    @pl.loop(0, epw // 16)
    def _add(g):
        sl = pl.ds(g * 16, 16)
        vals[sl] = vals[sl] + ev[sl]

    pltpu.sync_copy(vals, out_hbm.at[pl.ds(base, epw)])


def kernel(patient_embeds, condition_embeds, edge_index, tte, scale, bias,
           time_coeff):
    n, d = patient_embeds.shape
    e = edge_index.shape[1]
    assert d == 128 and n == 10000 and e % (NW * GC) == 0

    pn, cnp, ex, fx = pl.pallas_call(
        functools.partial(_prep_body, n),
        out_shape=(
            jax.ShapeDtypeStruct((n, d), jnp.float32),
            jax.ShapeDtypeStruct((NPAD, d), jnp.float32),
            jax.ShapeDtypeStruct((e // 128, 128), jnp.float32),
            jax.ShapeDtypeStruct((e // 128, 128), jnp.int32),
        ),
        in_specs=[
            pl.BlockSpec(memory_space=pltpu.VMEM),
            pl.BlockSpec(memory_space=pltpu.VMEM),
            pl.BlockSpec(memory_space=pltpu.VMEM),
            pl.BlockSpec(memory_space=pltpu.VMEM),
            pl.BlockSpec(memory_space=pltpu.VMEM),
            pl.BlockSpec(memory_space=pltpu.SMEM),
            pl.BlockSpec(memory_space=pltpu.SMEM),
            pl.BlockSpec(memory_space=pltpu.SMEM),
        ],
    )(patient_embeds, condition_embeds,
      tte.reshape(e // 128, 128),
      edge_index[0].reshape(e // 128, 128),
      edge_index[1].reshape(e // 128, 128),
      scale.reshape(1, 1).astype(jnp.float32),
      bias.reshape(1, 1).astype(jnp.float32),
      time_coeff.reshape(1, 1).astype(jnp.float32))
    ex = ex.reshape(e)
    fx = fx.reshape(e)

    bi, bj = 1000, 1024
    nkc = NPAD // d
    s = pl.pallas_call(
        _mm_body,
        grid=(n // bi, NPAD // bj),
        out_shape=jax.ShapeDtypeStruct((nkc, n, d), jnp.float32),
        in_specs=[
            pl.BlockSpec((bi, d), lambda i, j: (i, 0)),
            pl.BlockSpec((bj, d), lambda i, j: (j, 0)),
        ],
        out_specs=pl.BlockSpec((bj // d, bi, d), lambda i, j: (j, i, 0)),
    )(pn, cnp)
    s_flat = s.reshape(nkc * n * d)

    epw = e // NW
    ng = epw // GC
    mesh = plsc.VectorSubcoreMesh(core_axis_name="c", subcore_axis_name="s")
    cp = pltpu.CompilerParams()
    if "needs_layout_passes" in pltpu.CompilerParams.__dataclass_fields__:
        cp = dataclasses.replace(cp, needs_layout_passes=False)
    sc = pl.kernel(
        functools.partial(_sc_body, epw, ng),
        out_type=jax.ShapeDtypeStruct((e,), jnp.float32),
        mesh=mesh,
        scratch_types=[
            pltpu.VMEM((epw,), jnp.int32),
            pltpu.VMEM((epw,), jnp.float32),
            pltpu.VMEM((epw,), jnp.float32),
            pltpu.SemaphoreType.DMA,
        ],
        compiler_params=cp,
    )
    return sc(s_flat, fx, ex)


# speedup vs baseline: 2.8224x; 1.0412x over previous
import dataclasses
import functools

import jax
import jax.numpy as jnp
from jax import lax
from jax.experimental import pallas as pl
from jax.experimental.pallas import tpu as pltpu
from jax.experimental.pallas import tpu_sc as plsc

EPS = 1e-8
NUM_CORES = 2
NUM_SUBCORES = 16
NW = NUM_CORES * NUM_SUBCORES
NPAD = 10240
GC = 80


def _prep_body(n_rows, p_ref, c_ref, t_ref, src_ref, dst_ref, s_ref, b_ref,
               tc_ref, pn_ref, cn_ref, ex_ref, fx_ref):
    scale = s_ref[0, 0]
    p = p_ref[...]
    pn = jnp.maximum(jnp.sqrt(jnp.sum(p * p, axis=1, keepdims=True)), EPS)
    pn_ref[...] = (p * (scale / pn)).astype(jnp.bfloat16)
    c = c_ref[...]
    cn = jnp.maximum(jnp.sqrt(jnp.sum(c * c, axis=1, keepdims=True)), EPS)
    n = c.shape[0]
    cn_ref[0:n, :] = (c / cn).astype(jnp.bfloat16)
    cn_ref[n:NPAD, :] = jnp.zeros((NPAD - n, c.shape[1]), jnp.bfloat16)
    t = t_ref[...]
    boost = jnp.where(t > 0, 1.0 / (t + 1.0), jnp.zeros_like(t))
    ex_ref[...] = tc_ref[0, 0] * boost + b_ref[0, 0]
    src = src_ref[...]
    dst = dst_ref[...]
    fx_ref[...] = ((dst >> 7) * n_rows + src) * 128 + (dst & 127)


def _mm_body(pn_ref, cn_ref, o_ref):
    res = lax.dot_general(
        pn_ref[...], cn_ref[...], (((1,), (1,)), ((), ())),
        preferred_element_type=jnp.float32)
    for kk in range(o_ref.shape[0]):
        o_ref[kk] = res[:, kk * 128:(kk + 1) * 128]


def _sc_body(epw, ng, s_hbm, fx_hbm, ex_hbm, out_hbm, fv, ev, vals, sem):
    wid = lax.axis_index("c") * NUM_SUBCORES + lax.axis_index("s")
    base = wid * epw

    pltpu.sync_copy(fx_hbm.at[pl.ds(base, epw)], fv)
    pltpu.sync_copy(ex_hbm.at[pl.ds(base, epw)], ev)

    @pl.loop(0, ng)
    def _issue(k):
        sl = pl.ds(k * GC, GC)
        pltpu.async_copy(s_hbm.at[fv.at[sl]], vals.at[sl], sem)

    @pl.loop(0, ng)
    def _drain(k):
        pltpu.make_async_copy(
            s_hbm.at[pl.ds(0, GC)], vals.at[pl.ds(0, GC)], sem).wait()

    @pl.loop(0, epw // 16)
    def _add(g):
        sl = pl.ds(g * 16, 16)
        vals[sl] = vals[sl] + ev[sl]

    pltpu.sync_copy(vals, out_hbm.at[pl.ds(base, epw)])


def kernel(patient_embeds, condition_embeds, edge_index, tte, scale, bias,
           time_coeff):
    n, d = patient_embeds.shape
    e = edge_index.shape[1]
    assert d == 128 and n == 10000 and e % (NW * GC) == 0

    pn, cnp, ex, fx = pl.pallas_call(
        functools.partial(_prep_body, n),
        out_shape=(
            jax.ShapeDtypeStruct((n, d), jnp.bfloat16),
            jax.ShapeDtypeStruct((NPAD, d), jnp.bfloat16),
            jax.ShapeDtypeStruct((e // 128, 128), jnp.float32),
            jax.ShapeDtypeStruct((e // 128, 128), jnp.int32),
        ),
        in_specs=[
            pl.BlockSpec(memory_space=pltpu.VMEM),
            pl.BlockSpec(memory_space=pltpu.VMEM),
            pl.BlockSpec(memory_space=pltpu.VMEM),
            pl.BlockSpec(memory_space=pltpu.VMEM),
            pl.BlockSpec(memory_space=pltpu.VMEM),
            pl.BlockSpec(memory_space=pltpu.SMEM),
            pl.BlockSpec(memory_space=pltpu.SMEM),
            pl.BlockSpec(memory_space=pltpu.SMEM),
        ],
    )(patient_embeds, condition_embeds,
      tte.reshape(e // 128, 128),
      edge_index[0].reshape(e // 128, 128),
      edge_index[1].reshape(e // 128, 128),
      scale.reshape(1, 1).astype(jnp.float32),
      bias.reshape(1, 1).astype(jnp.float32),
      time_coeff.reshape(1, 1).astype(jnp.float32))
    ex = ex.reshape(e)
    fx = fx.reshape(e)

    bi, bj = 1000, 1024
    nkc = NPAD // d
    s = pl.pallas_call(
        _mm_body,
        grid=(n // bi, NPAD // bj),
        out_shape=jax.ShapeDtypeStruct((nkc, n, d), jnp.float32),
        in_specs=[
            pl.BlockSpec((bi, d), lambda i, j: (i, 0)),
            pl.BlockSpec((bj, d), lambda i, j: (j, 0)),
        ],
        out_specs=pl.BlockSpec((bj // d, bi, d), lambda i, j: (j, i, 0)),
    )(pn, cnp)
    s_flat = s.reshape(nkc * n * d)

    epw = e // NW
    ng = epw // GC
    mesh = plsc.VectorSubcoreMesh(core_axis_name="c", subcore_axis_name="s")
    cp = pltpu.CompilerParams()
    if "needs_layout_passes" in pltpu.CompilerParams.__dataclass_fields__:
        cp = dataclasses.replace(cp, needs_layout_passes=False)
    sc = pl.kernel(
        functools.partial(_sc_body, epw, ng),
        out_type=jax.ShapeDtypeStruct((e,), jnp.float32),
        mesh=mesh,
        scratch_types=[
            pltpu.VMEM((epw,), jnp.int32),
            pltpu.VMEM((epw,), jnp.float32),
            pltpu.VMEM((epw,), jnp.float32),
            pltpu.SemaphoreType.DMA,
        ],
        compiler_params=cp,
    )
    return sc(s_flat, fx, ex)


# speedup vs baseline: 3.1988x; 1.1334x over previous
import dataclasses
import functools

import jax
import jax.numpy as jnp
from jax import lax
from jax.experimental import pallas as pl
from jax.experimental.pallas import tpu as pltpu
from jax.experimental.pallas import tpu_sc as plsc

EPS = 1e-8
NUM_CORES = 2
NUM_SUBCORES = 16
NW = NUM_CORES * NUM_SUBCORES
NPAD = 10240
GC = 80


def _prep_body(n_rows, p_ref, c_ref, t_ref, src_ref, dst_ref, s_ref, b_ref,
               tc_ref, pn_ref, cn_ref, ex_ref, fx_ref):
    scale = s_ref[0, 0]
    p = p_ref[...]
    pn = jnp.maximum(jnp.sqrt(jnp.sum(p * p, axis=1, keepdims=True)), EPS)
    pn_ref[...] = (p * (scale / pn)).astype(jnp.bfloat16)
    c = c_ref[...]
    cn = jnp.maximum(jnp.sqrt(jnp.sum(c * c, axis=1, keepdims=True)), EPS)
    n = c.shape[0]
    cn_ref[0:n, :] = (c / cn).astype(jnp.bfloat16)
    cn_ref[n:NPAD, :] = jnp.zeros((NPAD - n, c.shape[1]), jnp.bfloat16)
    t = t_ref[...]
    boost = jnp.where(t > 0, 1.0 / (t + 1.0), jnp.zeros_like(t))
    ex_ref[...] = tc_ref[0, 0] * boost + b_ref[0, 0]
    src = src_ref[...]
    dst = dst_ref[...]
    fx_ref[...] = ((dst >> 8) * n_rows + src) * 128 + (dst & 127)


def _mm_body(pn_ref, cn_ref, o_ref):
    res = lax.dot_general(
        pn_ref[...], cn_ref[...], (((1,), (1,)), ((), ())),
        preferred_element_type=jnp.float32)

    def bf16_bits(x):
        u = pltpu.bitcast(x, jnp.int32)
        return (u + 0x7FFF + ((u >> 16) & 1)) >> 16

    for kk in range(o_ref.shape[0]):
        lo = bf16_bits(res[:, kk * 256:kk * 256 + 128])
        hi = bf16_bits(res[:, kk * 256 + 128:kk * 256 + 256])
        o_ref[kk] = (lo & 0xFFFF) | (hi << 16)


def _post_body(w_ref, dst_ref, ex_ref, o_ref):
    w = w_ref[...]
    sel_hi = (dst_ref[...] & 128) > 0
    bits = jnp.where(sel_hi, w & jnp.int32(-65536), w << 16)
    o_ref[...] = pltpu.bitcast(bits, jnp.float32) + ex_ref[...]


def _sc_body(epw, ng, s_hbm, fx_hbm, out_hbm, fv, vals, sem):
    wid = lax.axis_index("c") * NUM_SUBCORES + lax.axis_index("s")
    base = wid * epw

    pltpu.sync_copy(fx_hbm.at[pl.ds(base, epw)], fv)

    @pl.loop(0, ng)
    def _issue(k):
        sl = pl.ds(k * GC, GC)
        pltpu.async_copy(s_hbm.at[fv.at[sl]], vals.at[sl], sem)

    @pl.loop(0, ng)
    def _drain(k):
        pltpu.make_async_copy(
            s_hbm.at[pl.ds(0, GC)], vals.at[pl.ds(0, GC)], sem).wait()

    pltpu.sync_copy(vals, out_hbm.at[pl.ds(base, epw)])


def kernel(patient_embeds, condition_embeds, edge_index, tte, scale, bias,
           time_coeff):
    n, d = patient_embeds.shape
    e = edge_index.shape[1]
    assert d == 128 and n == 10000 and e % (NW * GC) == 0

    dst2d = edge_index[1].reshape(e // 128, 128)
    pn, cnp, ex, fx = pl.pallas_call(
        functools.partial(_prep_body, n),
        out_shape=(
            jax.ShapeDtypeStruct((n, d), jnp.bfloat16),
            jax.ShapeDtypeStruct((NPAD, d), jnp.bfloat16),
            jax.ShapeDtypeStruct((e // 128, 128), jnp.float32),
            jax.ShapeDtypeStruct((e // 128, 128), jnp.int32),
        ),
        in_specs=[
            pl.BlockSpec(memory_space=pltpu.VMEM),
            pl.BlockSpec(memory_space=pltpu.VMEM),
            pl.BlockSpec(memory_space=pltpu.VMEM),
            pl.BlockSpec(memory_space=pltpu.VMEM),
            pl.BlockSpec(memory_space=pltpu.VMEM),
            pl.BlockSpec(memory_space=pltpu.SMEM),
            pl.BlockSpec(memory_space=pltpu.SMEM),
            pl.BlockSpec(memory_space=pltpu.SMEM),
        ],
    )(patient_embeds, condition_embeds,
      tte.reshape(e // 128, 128),
      edge_index[0].reshape(e // 128, 128),
      dst2d,
      scale.reshape(1, 1).astype(jnp.float32),
      bias.reshape(1, 1).astype(jnp.float32),
      time_coeff.reshape(1, 1).astype(jnp.float32))
    fx = fx.reshape(e)

    bi, bj = 1000, 1024
    nwc = NPAD // 256
    s = pl.pallas_call(
        _mm_body,
        grid=(n // bi, NPAD // bj),
        out_shape=jax.ShapeDtypeStruct((nwc, n, d), jnp.int32),
        in_specs=[
            pl.BlockSpec((bi, d), lambda i, j: (i, 0)),
            pl.BlockSpec((bj, d), lambda i, j: (j, 0)),
        ],
        out_specs=pl.BlockSpec((bj // 256, bi, d), lambda i, j: (j, i, 0)),
    )(pn, cnp)
    s_flat = s.reshape(nwc * n * d)

    epw = e // NW
    ng = epw // GC
    mesh = plsc.VectorSubcoreMesh(core_axis_name="c", subcore_axis_name="s")
    cp = pltpu.CompilerParams()
    if "needs_layout_passes" in pltpu.CompilerParams.__dataclass_fields__:
        cp = dataclasses.replace(cp, needs_layout_passes=False)
    sc = pl.kernel(
        functools.partial(_sc_body, epw, ng),
        out_type=jax.ShapeDtypeStruct((e,), jnp.int32),
        mesh=mesh,
        scratch_types=[
            pltpu.VMEM((epw,), jnp.int32),
            pltpu.VMEM((epw,), jnp.int32),
            pltpu.SemaphoreType.DMA,
        ],
        compiler_params=cp,
    )
    words = sc(s_flat, fx)

    logits = pl.pallas_call(
        _post_body,
        out_shape=jax.ShapeDtypeStruct((e // 128, 128), jnp.float32),
        in_specs=[
            pl.BlockSpec(memory_space=pltpu.VMEM),
            pl.BlockSpec(memory_space=pltpu.VMEM),
            pl.BlockSpec(memory_space=pltpu.VMEM),
        ],
    )(words.reshape(e // 128, 128), dst2d, ex)
    return logits.reshape(e)


# speedup vs baseline: 3.3975x; 1.0621x over previous
import dataclasses
import functools

import jax
import jax.numpy as jnp
from jax import lax
from jax.experimental import pallas as pl
from jax.experimental.pallas import tpu as pltpu
from jax.experimental.pallas import tpu_sc as plsc

EPS = 1e-8
NUM_CORES = 2
NUM_SUBCORES = 16
NW = NUM_CORES * NUM_SUBCORES
NPAD = 10240
GC = 80


def _prep_body(n_rows, p_ref, c_ref, t_ref, src_ref, dst_ref, s_ref, b_ref,
               tc_ref, pn_ref, cn_ref, ex_ref, fx_ref):
    scale = s_ref[0, 0]
    p = p_ref[...]
    pn = jnp.maximum(jnp.sqrt(jnp.sum(p * p, axis=1, keepdims=True)), EPS)
    pn_ref[...] = (p * (scale / pn)).astype(jnp.bfloat16)
    c = c_ref[...]
    cn = jnp.maximum(jnp.sqrt(jnp.sum(c * c, axis=1, keepdims=True)), EPS)
    n = c.shape[0]
    cn_ref[0:n, :] = (c / cn).astype(jnp.bfloat16)
    cn_ref[n:NPAD, :] = jnp.zeros((NPAD - n, c.shape[1]), jnp.bfloat16)
    t = t_ref[...]
    boost = jnp.where(t > 0, 1.0 / (t + 1.0), jnp.zeros_like(t))
    ex_ref[...] = tc_ref[0, 0] * boost + b_ref[0, 0]
    src = src_ref[...]
    dst = dst_ref[...]
    fx_ref[...] = ((dst >> 8) * n_rows + src) * 128 + (dst & 127)


def _mm_body(pn_ref, cn_ref, o_ref):
    res = lax.dot_general(
        pn_ref[...], cn_ref[...], (((1,), (1,)), ((), ())),
        preferred_element_type=jnp.float32)

    for kk in range(o_ref.shape[0]):
        ul = pltpu.bitcast(res[:, kk * 256:kk * 256 + 128], jnp.uint32)
        uh = pltpu.bitcast(res[:, kk * 256 + 128:kk * 256 + 256], jnp.uint32)
        lo = (ul + 0x8000) >> 16
        hi = (uh + 0x8000) & jnp.uint32(0xFFFF0000)
        o_ref[kk] = pltpu.bitcast(lo | hi, jnp.int32)


def _post_body(w_ref, dst_ref, ex_ref, o_ref):
    w = w_ref[...]
    sel_hi = (dst_ref[...] & 128) > 0
    bits = jnp.where(sel_hi, w & jnp.int32(-65536), w << 16)
    o_ref[...] = pltpu.bitcast(bits, jnp.float32) + ex_ref[...]


def _sc_body(epw, ng, s_hbm, fx_hbm, out_hbm, fv, vals, sem):
    wid = lax.axis_index("c") * NUM_SUBCORES + lax.axis_index("s")
    base = wid * epw

    pltpu.sync_copy(fx_hbm.at[pl.ds(base, epw)], fv)

    @pl.loop(0, ng)
    def _issue(k):
        sl = pl.ds(k * GC, GC)
        pltpu.async_copy(s_hbm.at[fv.at[sl]], vals.at[sl], sem)

    @pl.loop(0, ng)
    def _drain(k):
        pltpu.make_async_copy(
            s_hbm.at[pl.ds(0, GC)], vals.at[pl.ds(0, GC)], sem).wait()

    pltpu.sync_copy(vals, out_hbm.at[pl.ds(base, epw)])


def kernel(patient_embeds, condition_embeds, edge_index, tte, scale, bias,
           time_coeff):
    n, d = patient_embeds.shape
    e = edge_index.shape[1]
    assert d == 128 and n == 10000 and e % (NW * GC) == 0

    dst2d = edge_index[1].reshape(e // 128, 128)
    pn, cnp, ex, fx = pl.pallas_call(
        functools.partial(_prep_body, n),
        out_shape=(
            jax.ShapeDtypeStruct((n, d), jnp.bfloat16),
            jax.ShapeDtypeStruct((NPAD, d), jnp.bfloat16),
            jax.ShapeDtypeStruct((e // 128, 128), jnp.float32),
            jax.ShapeDtypeStruct((e // 128, 128), jnp.int32),
        ),
        in_specs=[
            pl.BlockSpec(memory_space=pltpu.VMEM),
            pl.BlockSpec(memory_space=pltpu.VMEM),
            pl.BlockSpec(memory_space=pltpu.VMEM),
            pl.BlockSpec(memory_space=pltpu.VMEM),
            pl.BlockSpec(memory_space=pltpu.VMEM),
            pl.BlockSpec(memory_space=pltpu.SMEM),
            pl.BlockSpec(memory_space=pltpu.SMEM),
            pl.BlockSpec(memory_space=pltpu.SMEM),
        ],
    )(patient_embeds, condition_embeds,
      tte.reshape(e // 128, 128),
      edge_index[0].reshape(e // 128, 128),
      dst2d,
      scale.reshape(1, 1).astype(jnp.float32),
      bias.reshape(1, 1).astype(jnp.float32),
      time_coeff.reshape(1, 1).astype(jnp.float32))
    fx = fx.reshape(e)

    bi, bj = 1000, 1024
    nwc = NPAD // 256
    s = pl.pallas_call(
        _mm_body,
        grid=(n // bi, NPAD // bj),
        out_shape=jax.ShapeDtypeStruct((nwc, n, d), jnp.int32),
        in_specs=[
            pl.BlockSpec((bi, d), lambda i, j: (i, 0)),
            pl.BlockSpec((bj, d), lambda i, j: (j, 0)),
        ],
        out_specs=pl.BlockSpec((bj // 256, bi, d), lambda i, j: (j, i, 0)),
    )(pn, cnp)
    s_flat = s.reshape(nwc * n * d)

    epw = e // NW
    ng = epw // GC
    mesh = plsc.VectorSubcoreMesh(core_axis_name="c", subcore_axis_name="s")
    cp = pltpu.CompilerParams()
    if "needs_layout_passes" in pltpu.CompilerParams.__dataclass_fields__:
        cp = dataclasses.replace(cp, needs_layout_passes=False)
    sc = pl.kernel(
        functools.partial(_sc_body, epw, ng),
        out_type=jax.ShapeDtypeStruct((e,), jnp.int32),
        mesh=mesh,
        scratch_types=[
            pltpu.VMEM((epw,), jnp.int32),
            pltpu.VMEM((epw,), jnp.int32),
            pltpu.SemaphoreType.DMA,
        ],
        compiler_params=cp,
    )
    words = sc(s_flat, fx)

    logits = pl.pallas_call(
        _post_body,
        out_shape=jax.ShapeDtypeStruct((e // 128, 128), jnp.float32),
        in_specs=[
            pl.BlockSpec(memory_space=pltpu.VMEM),
            pl.BlockSpec(memory_space=pltpu.VMEM),
            pl.BlockSpec(memory_space=pltpu.VMEM),
        ],
    )(words.reshape(e // 128, 128), dst2d, ex)
    return logits.reshape(e)


# speedup vs baseline: 4.1182x; 1.2121x over previous
import dataclasses
import functools

import jax
import jax.numpy as jnp
from jax import lax
from jax.experimental import pallas as pl
from jax.experimental.pallas import tpu as pltpu
from jax.experimental.pallas import tpu_sc as plsc

EPS = 1e-8
NUM_CORES = 2
NUM_SUBCORES = 16
NW = NUM_CORES * NUM_SUBCORES
NPAD = 10240
GC = 80


def _prep_body(n_rows, p_ref, c_ref, t_ref, src_ref, dst_ref, s_ref, b_ref,
               tc_ref, pn_ref, cn_ref, ex_ref, fx_ref):
    scale = s_ref[0, 0]
    p = p_ref[...]
    pn = jnp.maximum(jnp.sqrt(jnp.sum(p * p, axis=1, keepdims=True)), EPS)
    pn_ref[...] = (p * (scale / pn)).astype(jnp.bfloat16)
    c = c_ref[...]
    cn = jnp.maximum(jnp.sqrt(jnp.sum(c * c, axis=1, keepdims=True)), EPS)
    n = c.shape[0]
    cn_ref[0:n, :] = (c / cn).astype(jnp.bfloat16)
    cn_ref[n:NPAD, :] = jnp.zeros((NPAD - n, c.shape[1]), jnp.bfloat16)
    t = t_ref[...]
    boost = jnp.where(t > 0, 1.0 / (t + 1.0), jnp.zeros_like(t))
    ex_ref[...] = tc_ref[0, 0] * boost + b_ref[0, 0]
    src = src_ref[...]
    dst = dst_ref[...]
    fx_ref[...] = ((dst >> 8) * n_rows + src) * 128 + (dst & 127)


def _mm_body(pn_ref, cn_ref, o_ref):
    res = lax.dot_general(
        pn_ref[...], cn_ref[...], (((1,), (1,)), ((), ())),
        preferred_element_type=jnp.float32)

    for kk in range(o_ref.shape[0]):
        ul = pltpu.bitcast(res[:, kk * 256:kk * 256 + 128], jnp.uint32)
        uh = pltpu.bitcast(res[:, kk * 256 + 128:kk * 256 + 256], jnp.uint32)
        lo = (ul + 0x8000) >> 16
        hi = (uh + 0x8000) & jnp.uint32(0xFFFF0000)
        o_ref[kk] = pltpu.bitcast(lo | hi, jnp.int32)


def _post_body(w_ref, dst_ref, ex_ref, o_ref):
    w = w_ref[...]
    sel_hi = (dst_ref[...] & 128) > 0
    bits = jnp.where(sel_hi, w & jnp.int32(-65536), w << 16)
    o_ref[...] = pltpu.bitcast(bits, jnp.float32) + ex_ref[...]


def _sc_body(epw, ng, s_hbm, fx_hbm, out_hbm, fv, vals, sem):
    wid = lax.axis_index("c") * NUM_SUBCORES + lax.axis_index("s")
    base = wid * epw

    pltpu.sync_copy(fx_hbm.at[pl.ds(base, epw)], fv)

    @pl.loop(0, ng)
    def _issue(k):
        sl = pl.ds(k * GC, GC)
        pltpu.async_copy(s_hbm.at[fv.at[sl]], vals.at[sl], sem)

    @pl.loop(0, ng)
    def _drain(k):
        pltpu.make_async_copy(
            s_hbm.at[pl.ds(0, GC)], vals.at[pl.ds(0, GC)], sem).wait()

    pltpu.sync_copy(vals, out_hbm.at[pl.ds(base, epw)])


def kernel(patient_embeds, condition_embeds, edge_index, tte, scale, bias,
           time_coeff):
    n, d = patient_embeds.shape
    e = edge_index.shape[1]
    assert d == 128 and n == 10000 and e % (NW * GC) == 0

    dst2d = edge_index[1].reshape(e // 128, 128)
    pn, cnp, ex, fx = pl.pallas_call(
        functools.partial(_prep_body, n),
        out_shape=(
            jax.ShapeDtypeStruct((n, d), jnp.bfloat16),
            jax.ShapeDtypeStruct((NPAD, d), jnp.bfloat16),
            jax.ShapeDtypeStruct((e // 128, 128), jnp.float32),
            jax.ShapeDtypeStruct((e // 128, 128), jnp.int32),
        ),
        in_specs=[
            pl.BlockSpec(memory_space=pltpu.VMEM),
            pl.BlockSpec(memory_space=pltpu.VMEM),
            pl.BlockSpec(memory_space=pltpu.VMEM),
            pl.BlockSpec(memory_space=pltpu.VMEM),
            pl.BlockSpec(memory_space=pltpu.VMEM),
            pl.BlockSpec(memory_space=pltpu.SMEM),
            pl.BlockSpec(memory_space=pltpu.SMEM),
            pl.BlockSpec(memory_space=pltpu.SMEM),
        ],
    )(patient_embeds, condition_embeds,
      tte.reshape(e // 128, 128),
      edge_index[0].reshape(e // 128, 128),
      dst2d,
      scale.reshape(1, 1).astype(jnp.float32),
      bias.reshape(1, 1).astype(jnp.float32),
      time_coeff.reshape(1, 1).astype(jnp.float32))
    fx = fx.reshape(e)

    bi, bj = 2000, 1024
    nwc = NPAD // 256
    s = pl.pallas_call(
        _mm_body,
        grid=(n // bi, NPAD // bj),
        out_shape=jax.ShapeDtypeStruct((nwc, n, d), jnp.int32),
        in_specs=[
            pl.BlockSpec((bi, d), lambda i, j: (i, 0)),
            pl.BlockSpec((bj, d), lambda i, j: (j, 0)),
        ],
        out_specs=pl.BlockSpec((bj // 256, bi, d), lambda i, j: (j, i, 0)),
    )(pn, cnp)
    s_flat = s.reshape(nwc * n * d)

    epw = e // NW
    ng = epw // GC
    mesh = plsc.VectorSubcoreMesh(core_axis_name="c", subcore_axis_name="s")
    cp = pltpu.CompilerParams()
    if "needs_layout_passes" in pltpu.CompilerParams.__dataclass_fields__:
        cp = dataclasses.replace(cp, needs_layout_passes=False)
    sc = pl.kernel(
        functools.partial(_sc_body, epw, ng),
        out_type=jax.ShapeDtypeStruct((e,), jnp.int32),
        mesh=mesh,
        scratch_types=[
            pltpu.VMEM((epw,), jnp.int32),
            pltpu.VMEM((epw,), jnp.int32),
            pltpu.SemaphoreType.DMA,
        ],
        compiler_params=cp,
    )
    words = sc(s_flat, fx)

    logits = pl.pallas_call(
        _post_body,
        out_shape=jax.ShapeDtypeStruct((e // 128, 128), jnp.float32),
        in_specs=[
            pl.BlockSpec(memory_space=pltpu.VMEM),
            pl.BlockSpec(memory_space=pltpu.VMEM),
            pl.BlockSpec(memory_space=pltpu.VMEM),
        ],
    )(words.reshape(e // 128, 128), dst2d, ex)
    return logits.reshape(e)


# speedup vs baseline: 4.5330x; 1.1007x over previous
import dataclasses
import functools

import jax
import jax.numpy as jnp
from jax import lax
from jax.experimental import pallas as pl
from jax.experimental.pallas import tpu as pltpu
from jax.experimental.pallas import tpu_sc as plsc

EPS = 1e-8
NUM_CORES = 2
NUM_SUBCORES = 16
NW = NUM_CORES * NUM_SUBCORES
NPAD = 10240
GC = 80


def _prep_body(n_rows, p_ref, c_ref, t_ref, src_ref, dst_ref, s_ref, b_ref,
               tc_ref, pn_ref, cn_ref, ex_ref, fx_ref):
    scale = s_ref[0, 0]
    p = p_ref[...]
    pn = jnp.maximum(jnp.sqrt(jnp.sum(p * p, axis=1, keepdims=True)), EPS)
    pn_ref[...] = (p * (scale / pn)).astype(jnp.bfloat16)
    c = c_ref[...]
    cn = jnp.maximum(jnp.sqrt(jnp.sum(c * c, axis=1, keepdims=True)), EPS)
    n = c.shape[0]
    cn_ref[0:n, :] = (c / cn).astype(jnp.bfloat16)
    cn_ref[n:NPAD, :] = jnp.zeros((NPAD - n, c.shape[1]), jnp.bfloat16)
    t = t_ref[...]
    boost = jnp.where(t > 0, 1.0 / (t + 1.0), jnp.zeros_like(t))
    ex_ref[...] = tc_ref[0, 0] * boost + b_ref[0, 0]
    src = src_ref[...]
    dst = dst_ref[...]
    fx_ref[...] = ((dst >> 8) * n_rows + src) * 128 + (dst & 127)


def _mm_body(pn_ref, cn_ref, o_ref):
    res = lax.dot_general(
        pn_ref[...], cn_ref[...], (((1,), (1,)), ((), ())),
        preferred_element_type=jnp.float32)

    for kk in range(o_ref.shape[0]):
        ul = pltpu.bitcast(res[:, kk * 256:kk * 256 + 128], jnp.uint32)
        uh = pltpu.bitcast(res[:, kk * 256 + 128:kk * 256 + 256], jnp.uint32)
        lo = (ul + 0x8000) >> 16
        hi = (uh + 0x8000) & jnp.uint32(0xFFFF0000)
        o_ref[kk] = pltpu.bitcast(lo | hi, jnp.int32)


def _post_body(w_ref, dst_ref, ex_ref, o_ref):
    w = w_ref[...]
    sel_hi = (dst_ref[...] & 128) > 0
    bits = jnp.where(sel_hi, w & jnp.int32(-65536), w << 16)
    o_ref[...] = pltpu.bitcast(bits, jnp.float32) + ex_ref[...]


def _sc_body(epw, ng, s_hbm, fx_hbm, out_hbm, fv, vals, sem):
    wid = lax.axis_index("c") * NUM_SUBCORES + lax.axis_index("s")
    base = wid * epw

    pltpu.sync_copy(fx_hbm.at[pl.ds(base, epw)], fv)

    @pl.loop(0, ng)
    def _issue(k):
        sl = pl.ds(k * GC, GC)
        pltpu.async_copy(s_hbm.at[fv.at[sl]], vals.at[sl], sem)

    @pl.loop(0, ng)
    def _drain(k):
        pltpu.make_async_copy(
            s_hbm.at[pl.ds(0, GC)], vals.at[pl.ds(0, GC)], sem).wait()

    pltpu.sync_copy(vals, out_hbm.at[pl.ds(base, epw)])


def kernel(patient_embeds, condition_embeds, edge_index, tte, scale, bias,
           time_coeff):
    n, d = patient_embeds.shape
    e = edge_index.shape[1]
    assert d == 128 and n == 10000 and e % (NW * GC) == 0

    dst2d = edge_index[1].reshape(e // 128, 128)
    pn, cnp, ex, fx = pl.pallas_call(
        functools.partial(_prep_body, n),
        out_shape=(
            jax.ShapeDtypeStruct((n, d), jnp.bfloat16),
            jax.ShapeDtypeStruct((NPAD, d), jnp.bfloat16),
            jax.ShapeDtypeStruct((e // 128, 128), jnp.float32),
            jax.ShapeDtypeStruct((e // 128, 128), jnp.int32),
        ),
        in_specs=[
            pl.BlockSpec(memory_space=pltpu.VMEM),
            pl.BlockSpec(memory_space=pltpu.VMEM),
            pl.BlockSpec(memory_space=pltpu.VMEM),
            pl.BlockSpec(memory_space=pltpu.VMEM),
            pl.BlockSpec(memory_space=pltpu.VMEM),
            pl.BlockSpec(memory_space=pltpu.SMEM),
            pl.BlockSpec(memory_space=pltpu.SMEM),
            pl.BlockSpec(memory_space=pltpu.SMEM),
        ],
    )(patient_embeds, condition_embeds,
      tte.reshape(e // 128, 128),
      edge_index[0].reshape(e // 128, 128),
      dst2d,
      scale.reshape(1, 1).astype(jnp.float32),
      bias.reshape(1, 1).astype(jnp.float32),
      time_coeff.reshape(1, 1).astype(jnp.float32))
    fx = fx.reshape(e)

    bi, bj = 2000, 2048
    nwc = NPAD // 256
    s = pl.pallas_call(
        _mm_body,
        grid=(n // bi, NPAD // bj),
        out_shape=jax.ShapeDtypeStruct((nwc, n, d), jnp.int32),
        in_specs=[
            pl.BlockSpec((bi, d), lambda i, j: (i, 0)),
            pl.BlockSpec((bj, d), lambda i, j: (j, 0)),
        ],
        out_specs=pl.BlockSpec((bj // 256, bi, d), lambda i, j: (j, i, 0)),
    )(pn, cnp)
    s_flat = s.reshape(nwc * n * d)

    epw = e // NW
    ng = epw // GC
    mesh = plsc.VectorSubcoreMesh(core_axis_name="c", subcore_axis_name="s")
    cp = pltpu.CompilerParams()
    if "needs_layout_passes" in pltpu.CompilerParams.__dataclass_fields__:
        cp = dataclasses.replace(cp, needs_layout_passes=False)
    sc = pl.kernel(
        functools.partial(_sc_body, epw, ng),
        out_type=jax.ShapeDtypeStruct((e,), jnp.int32),
        mesh=mesh,
        scratch_types=[
            pltpu.VMEM((epw,), jnp.int32),
            pltpu.VMEM((epw,), jnp.int32),
            pltpu.SemaphoreType.DMA,
        ],
        compiler_params=cp,
    )
    words = sc(s_flat, fx)

    logits = pl.pallas_call(
        _post_body,
        out_shape=jax.ShapeDtypeStruct((e // 128, 128), jnp.float32),
        in_specs=[
            pl.BlockSpec(memory_space=pltpu.VMEM),
            pl.BlockSpec(memory_space=pltpu.VMEM),
            pl.BlockSpec(memory_space=pltpu.VMEM),
        ],
    )(words.reshape(e // 128, 128), dst2d, ex)
    return logits.reshape(e)
